# Initial kernel scaffold; baseline (speedup 1.0000x reference)
import jax
import jax.numpy as jnp
from jax.experimental import pallas as pl


def kernel(x, edge_index, W1, b1, W2, b2, W3, b3, g1, be1, g2, be2, g3, be3, Wf, bf):
    raise NotImplementedError("write your pallas kernel here")


# speedup vs baseline: 7.5636x; 7.5636x over previous
import functools

import jax
import jax.numpy as jnp
from jax import lax
from jax.experimental import pallas as pl
from jax.experimental.pallas import tpu as pltpu
from jax.experimental.pallas import tpu_sc as plsc

N_NODES = 10000
N_EDGES = 320000
D_FEAT = 128

NC = 2
NS = 16
NW = NC * NS

CHUNK = 128
N_PAD = 10112
ROWS_PT = N_PAD // NS
E_PAD = 327680
CHUNKS_PW = E_PAD // (CHUNK * NW)
IDXROWS_PW = CHUNKS_PW
DUMMY_COL = N_PAD - 8

_mesh = plsc.VectorSubcoreMesh(core_axis_name="c", subcore_axis_name="s")


def _memset2d(ref, nrows, ncols):
    zero16 = jnp.zeros((16,), jnp.float32)

    def row_body(r, _):
        for cc in range(ncols // 16):
            ref[r, pl.ds(cc * 16, 16)] = zero16
        return 0

    lax.fori_loop(0, nrows, row_body, 0)


@functools.partial(
    pl.kernel,
    mesh=_mesh,
    out_type=jax.ShapeDtypeStruct((NC, N_PAD, 16), jnp.float32),
    scratch_types=[
        pltpu.VMEM((IDXROWS_PW, CHUNK), jnp.int32),
        pltpu.VMEM((CHUNK, 16), jnp.float32),
        pltpu.VMEM_SHARED((N_PAD, 16), jnp.float32),
    ],
)
def _hist(col_hbm, out_hbm, cidx, ones_v, acc):
    c = lax.axis_index("c")
    s = lax.axis_index("s")
    wid = c * NS + s

    _memset2d(ones_v, CHUNK, 16)
    for r in range(ROWS_PT // CHUNK):
        pltpu.sync_copy(ones_v.at[pl.ds(0, CHUNK)],
                        acc.at[pl.ds(s * ROWS_PT + r * CHUNK, CHUNK)])
    rem = ROWS_PT % CHUNK
    pltpu.sync_copy(ones_v.at[pl.ds(0, rem)],
                    acc.at[pl.ds(s * ROWS_PT + (ROWS_PT // CHUNK) * CHUNK, rem)])

    one16 = jnp.ones((16,), jnp.float32)

    def fill_body(r, _):
        ones_v[r, pl.ds(0, 16)] = one16
        return 0

    lax.fori_loop(0, CHUNK, fill_body, 0)
    pltpu.sync_copy(col_hbm.at[pl.ds(wid * IDXROWS_PW, IDXROWS_PW)], cidx)
    plsc.subcore_barrier()

    def chunk_body(j, _):
        pltpu.sync_copy(ones_v, acc.at[cidx.at[j]], add=True)
        return 0

    lax.fori_loop(0, CHUNKS_PW, chunk_body, 0)
    plsc.subcore_barrier()
    pltpu.sync_copy(acc.at[pl.ds(s * ROWS_PT, ROWS_PT)],
                    out_hbm.at[c, pl.ds(s * ROWS_PT, ROWS_PT)])


SUP = 8
NSUP = CHUNKS_PW // SUP


@functools.partial(
    pl.kernel,
    mesh=_mesh,
    out_type=jax.ShapeDtypeStruct((NC, N_PAD, D_FEAT), jnp.float32),
    scratch_types=[
        pltpu.VMEM((2, SUP, CHUNK), jnp.int32),
        pltpu.VMEM((2, SUP, CHUNK), jnp.int32),
        pltpu.VMEM((CHUNK, D_FEAT), jnp.float32),
        pltpu.VMEM((CHUNK, D_FEAT), jnp.float32),
        pltpu.VMEM_SHARED((N_PAD, D_FEAT), jnp.float32),
        pltpu.SemaphoreType.DMA,
        pltpu.SemaphoreType.DMA,
        pltpu.SemaphoreType.DMA,
    ],
)
def _agg(y_hbm, row_hbm, col_hbm, out_hbm, ridx, cidx, bufa, bufb, acc,
         sema, semb, semr):
    c = lax.axis_index("c")
    s = lax.axis_index("s")
    wid = c * NS + s
    base = wid * IDXROWS_PW

    _memset2d(bufa, CHUNK, D_FEAT)
    for r in range(ROWS_PT // CHUNK):
        pltpu.sync_copy(bufa, acc.at[pl.ds(s * ROWS_PT + r * CHUNK, CHUNK)])
    rem = ROWS_PT % CHUNK
    pltpu.sync_copy(bufa.at[pl.ds(0, rem)],
                    acc.at[pl.ds(s * ROWS_PT + (ROWS_PT // CHUNK) * CHUNK, rem)])
    plsc.subcore_barrier()

    pltpu.sync_copy(row_hbm.at[pl.ds(base, SUP)], ridx.at[0])
    pltpu.sync_copy(col_hbm.at[pl.ds(base, SUP)], cidx.at[0])
    pltpu.async_copy(row_hbm.at[pl.ds(base + SUP, SUP)], ridx.at[1], semr)
    pltpu.async_copy(col_hbm.at[pl.ds(base + SUP, SUP)], cidx.at[1], semr)
    pltpu.async_copy(y_hbm.at[ridx.at[0, 0]], bufa, sema)

    def super_body(g, _):
        slot = g % 2
        nslot = (g + 1) % 2
        for j in range(SUP):
            cur, csem = (bufa, sema) if j % 2 == 0 else (bufb, semb)
            nxt, nsem = (bufb, semb) if j % 2 == 0 else (bufa, sema)
            if j + 1 < SUP:
                pltpu.async_copy(y_hbm.at[ridx.at[slot, j + 1]], nxt, nsem)
            else:
                @pl.when(g < NSUP - 1)
                def _():
                    nb = (base + (g + 1) * SUP,)
                    pltpu.make_async_copy(
                        row_hbm.at[pl.ds(nb[0], SUP)], ridx.at[nslot],
                        semr).wait()
                    pltpu.make_async_copy(
                        col_hbm.at[pl.ds(nb[0], SUP)], cidx.at[nslot],
                        semr).wait()
                    pltpu.async_copy(y_hbm.at[ridx.at[nslot, 0]], nxt, nsem)
            pltpu.make_async_copy(y_hbm.at[ridx.at[slot, j]], cur, csem).wait()
            pltpu.sync_copy(cur, acc.at[cidx.at[slot, j]], add=True)
            if j == SUP - 1:
                @pl.when(g < NSUP - 2)
                def _():
                    b2 = base + (g + 2) * SUP
                    pltpu.async_copy(row_hbm.at[pl.ds(b2, SUP)],
                                     ridx.at[slot], semr)
                    pltpu.async_copy(col_hbm.at[pl.ds(b2, SUP)],
                                     cidx.at[slot], semr)
        return 0

    lax.fori_loop(0, NSUP, super_body, 0)

    plsc.subcore_barrier()
    pltpu.sync_copy(acc.at[pl.ds(s * ROWS_PT, ROWS_PT)],
                    out_hbm.at[c, pl.ds(s * ROWS_PT, ROWS_PT)])


def _row_mask():
    rid = lax.broadcasted_iota(jnp.int32, (N_PAD, 1), 0)
    return rid < N_NODES


def _prep_body(x_ref, w_ref, degp_ref, dinv_ref, y_ref):
    deg = degp_ref[0, :, 0:1] + degp_ref[1, :, 0:1] + 1.0
    dinv = jnp.where(_row_mask(), lax.rsqrt(deg), 0.0)
    dinv_ref[...] = dinv
    xw = jnp.dot(x_ref[...], w_ref[...], preferred_element_type=jnp.float32)
    y_ref[...] = dinv * xw


def _bn_relu(aggp_ref, y_ref, dinv_ref, b_ref, g_ref, be_ref):
    mask = _row_mask()
    dinv = dinv_ref[...]
    h = dinv * (aggp_ref[0] + aggp_ref[1] + y_ref[...]) + b_ref[...]
    h = jnp.where(mask, h, 0.0)
    mu = jnp.sum(h, axis=0, keepdims=True) * (1.0 / N_NODES)
    d = jnp.where(mask, h - mu, 0.0)
    var = jnp.sum(d * d, axis=0, keepdims=True) * (1.0 / N_NODES)
    hn = g_ref[...] * d * lax.rsqrt(var + 1e-5) + be_ref[...]
    return jnp.maximum(jnp.where(mask, hn, 0.0), 0.0)


def _mid_body(aggp_ref, y_ref, dinv_ref, b_ref, g_ref, be_ref, w_ref, y2_ref):
    hn = _bn_relu(aggp_ref, y_ref, dinv_ref, b_ref, g_ref, be_ref)
    xw2 = jnp.dot(hn, w_ref[...], preferred_element_type=jnp.float32)
    y2_ref[...] = dinv_ref[...] * xw2


def _fin_body(aggp_ref, y_ref, dinv_ref, b_ref, g_ref, be_ref, wf_ref, bf_ref,
              out_ref):
    hn = _bn_relu(aggp_ref, y_ref, dinv_ref, b_ref, g_ref, be_ref)
    logits = jnp.dot(hn, wf_ref[...], preferred_element_type=jnp.float32)
    logits = logits + bf_ref[...]
    m = jnp.max(logits, axis=-1, keepdims=True)
    e = jnp.exp(logits - m)
    out_ref[...] = e / jnp.sum(e, axis=-1, keepdims=True)


_prep = pl.pallas_call(
    _prep_body,
    out_shape=(
        jax.ShapeDtypeStruct((N_PAD, 1), jnp.float32),
        jax.ShapeDtypeStruct((N_PAD, D_FEAT), jnp.float32),
    ),
)

_mid = pl.pallas_call(
    _mid_body,
    out_shape=jax.ShapeDtypeStruct((N_PAD, D_FEAT), jnp.float32),
)

_fin = pl.pallas_call(
    _fin_body,
    out_shape=jax.ShapeDtypeStruct((N_PAD, 10), jnp.float32),
)


def kernel(x, edge_index, W1, b1, W2, b2, W3, b3, g1, be1, g2, be2, g3, be3,
           Wf, bf):
    row = edge_index[0]
    col = edge_index[1]
    pad = E_PAD - N_EDGES
    rp = jnp.concatenate([row, jnp.zeros((pad,), jnp.int32)])
    cp = jnp.concatenate([col, jnp.full((pad,), DUMMY_COL, jnp.int32)])
    rp = rp.reshape(E_PAD // CHUNK, CHUNK)
    cp = cp.reshape(E_PAD // CHUNK, CHUNK)
    x_p = jnp.pad(x, ((0, N_PAD - N_NODES), (0, 0)))

    degp = _hist(cp)
    dinv, y1 = _prep(x_p, W1, degp)
    agg1 = _agg(y1, rp, cp)
    y2 = _mid(agg1, y1, dinv, b1.reshape(1, -1), g1.reshape(1, -1),
              be1.reshape(1, -1), W2)
    agg2 = _agg(y2, rp, cp)
    y3 = _mid(agg2, y2, dinv, b2.reshape(1, -1), g2.reshape(1, -1),
              be2.reshape(1, -1), W3)
    agg3 = _agg(y3, rp, cp)
    probs = _fin(agg3, y3, dinv, b3.reshape(1, -1), g3.reshape(1, -1),
                 be3.reshape(1, -1), Wf, bf.reshape(1, -1))
    return probs[:N_NODES]


# speedup vs baseline: 8.1755x; 1.0809x over previous
import functools

import jax
import jax.numpy as jnp
from jax import lax
from jax.experimental import pallas as pl
from jax.experimental.pallas import tpu as pltpu
from jax.experimental.pallas import tpu_sc as plsc

N_NODES = 10000
N_EDGES = 320000
D_FEAT = 128

NC = 2
NS = 16
NW = NC * NS

CHUNK = 128
N_PAD = 10112
ROWS_PT = N_PAD // NS
E_PAD = 327680
CHUNKS_PW = E_PAD // (CHUNK * NW)
IDXROWS_PW = CHUNKS_PW
DUMMY_COL = N_PAD - 8

_mesh = plsc.VectorSubcoreMesh(core_axis_name="c", subcore_axis_name="s")


def _memset2d(ref, nrows, ncols):
    zero16 = jnp.zeros((16,), jnp.float32)

    def row_body(r, _):
        for cc in range(ncols // 16):
            ref[r, pl.ds(cc * 16, 16)] = zero16
        return 0

    lax.fori_loop(0, nrows, row_body, 0)


@functools.partial(
    pl.kernel,
    mesh=_mesh,
    out_type=jax.ShapeDtypeStruct((NC, N_PAD, 16), jnp.float32),
    scratch_types=[
        pltpu.VMEM((IDXROWS_PW, CHUNK), jnp.int32),
        pltpu.VMEM((CHUNK, 16), jnp.float32),
        pltpu.VMEM_SHARED((N_PAD, 16), jnp.float32),
    ],
)
def _hist(col_hbm, out_hbm, cidx, ones_v, acc):
    c = lax.axis_index("c")
    s = lax.axis_index("s")
    wid = c * NS + s

    _memset2d(ones_v, CHUNK, 16)
    for r in range(ROWS_PT // CHUNK):
        pltpu.sync_copy(ones_v.at[pl.ds(0, CHUNK)],
                        acc.at[pl.ds(s * ROWS_PT + r * CHUNK, CHUNK)])
    rem = ROWS_PT % CHUNK
    pltpu.sync_copy(ones_v.at[pl.ds(0, rem)],
                    acc.at[pl.ds(s * ROWS_PT + (ROWS_PT // CHUNK) * CHUNK, rem)])

    one16 = jnp.ones((16,), jnp.float32)

    def fill_body(r, _):
        ones_v[r, pl.ds(0, 16)] = one16
        return 0

    lax.fori_loop(0, CHUNK, fill_body, 0)
    pltpu.sync_copy(col_hbm.at[pl.ds(wid * IDXROWS_PW, IDXROWS_PW)], cidx)
    plsc.subcore_barrier()

    def chunk_body(j, _):
        pltpu.sync_copy(ones_v, acc.at[cidx.at[j]], add=True)
        return 0

    lax.fori_loop(0, CHUNKS_PW, chunk_body, 0)
    plsc.subcore_barrier()
    pltpu.sync_copy(acc.at[pl.ds(s * ROWS_PT, ROWS_PT)],
                    out_hbm.at[c, pl.ds(s * ROWS_PT, ROWS_PT)])


SUP = 8
NSUP = CHUNKS_PW // SUP


@functools.partial(
    pl.kernel,
    mesh=_mesh,
    out_type=jax.ShapeDtypeStruct((NC, N_PAD, D_FEAT), jnp.float32),
    scratch_types=[
        pltpu.VMEM((2, SUP, CHUNK), jnp.int32),
        pltpu.VMEM((2, SUP, CHUNK), jnp.int32),
        pltpu.VMEM((CHUNK, D_FEAT), jnp.float32),
        pltpu.VMEM((CHUNK, D_FEAT), jnp.float32),
        pltpu.VMEM_SHARED((N_PAD, D_FEAT), jnp.float32),
        pltpu.SemaphoreType.DMA,
        pltpu.SemaphoreType.DMA,
        pltpu.SemaphoreType.DMA,
        pltpu.SemaphoreType.DMA,
        pltpu.SemaphoreType.DMA,
    ],
)
def _agg(y_hbm, row_hbm, col_hbm, out_hbm, ridx, cidx, bufa, bufb, acc,
         sema, semb, semr, sca, scb):
    c = lax.axis_index("c")
    s = lax.axis_index("s")
    wid = c * NS + s
    base = wid * IDXROWS_PW

    _memset2d(bufa, CHUNK, D_FEAT)
    for r in range(ROWS_PT // CHUNK):
        pltpu.sync_copy(bufa, acc.at[pl.ds(s * ROWS_PT + r * CHUNK, CHUNK)])
    rem = ROWS_PT % CHUNK
    pltpu.sync_copy(bufa.at[pl.ds(0, rem)],
                    acc.at[pl.ds(s * ROWS_PT + (ROWS_PT // CHUNK) * CHUNK, rem)])
    plsc.subcore_barrier()

    pltpu.sync_copy(row_hbm.at[pl.ds(base, SUP)], ridx.at[0])
    pltpu.sync_copy(col_hbm.at[pl.ds(base, SUP)], cidx.at[0])
    pltpu.async_copy(row_hbm.at[pl.ds(base + SUP, SUP)], ridx.at[1], semr)
    pltpu.async_copy(col_hbm.at[pl.ds(base + SUP, SUP)], cidx.at[1], semr)
    pltpu.async_copy(y_hbm.at[ridx.at[0, 0]], bufa, sema)

    def super_body(g, _):
        slot = g % 2
        nslot = (g + 1) % 2
        for j in range(SUP):
            cur, gcur = (bufa, sema) if j % 2 == 0 else (bufb, semb)
            nxt, gnxt = (bufb, semb) if j % 2 == 0 else (bufa, sema)
            scur = sca if j % 2 == 0 else scb
            snxt = scb if j % 2 == 0 else sca
            if j == 0:
                @pl.when(g > 0)
                def _():
                    pltpu.make_async_copy(
                        nxt, acc.at[cidx.at[nslot, SUP - 1]], snxt).wait()

                @pl.when(jnp.logical_and(g > 0, g < NSUP - 1))
                def _():
                    b1 = base + (g + 1) * SUP
                    pltpu.async_copy(row_hbm.at[pl.ds(b1, SUP)],
                                     ridx.at[nslot], semr)
                    pltpu.async_copy(col_hbm.at[pl.ds(b1, SUP)],
                                     cidx.at[nslot], semr)
            else:
                pltpu.make_async_copy(
                    nxt, acc.at[cidx.at[slot, j - 1]], snxt).wait()
            if j + 1 < SUP:
                pltpu.async_copy(y_hbm.at[ridx.at[slot, j + 1]], nxt, gnxt)
            else:
                @pl.when(g < NSUP - 1)
                def _():
                    nb = base + (g + 1) * SUP
                    pltpu.make_async_copy(
                        row_hbm.at[pl.ds(nb, SUP)], ridx.at[nslot],
                        semr).wait()
                    pltpu.make_async_copy(
                        col_hbm.at[pl.ds(nb, SUP)], cidx.at[nslot],
                        semr).wait()
                    pltpu.async_copy(y_hbm.at[ridx.at[nslot, 0]], nxt, gnxt)
            pltpu.make_async_copy(y_hbm.at[ridx.at[slot, j]], cur, gcur).wait()
            pltpu.async_copy(cur, acc.at[cidx.at[slot, j]], scur, add=True)
        return 0

    lax.fori_loop(0, NSUP, super_body, 0)
    pltpu.make_async_copy(bufb, acc.at[cidx.at[1, SUP - 1]], scb).wait()

    plsc.subcore_barrier()
    pltpu.sync_copy(acc.at[pl.ds(s * ROWS_PT, ROWS_PT)],
                    out_hbm.at[c, pl.ds(s * ROWS_PT, ROWS_PT)])


def _prep_body(x_ref, w_ref, degp_ref, dinv_ref, y_ref):
    deg = degp_ref[0, :N_NODES, 0:1] + degp_ref[1, :N_NODES, 0:1] + 1.0
    dinv = lax.rsqrt(deg)
    dinv_ref[...] = dinv
    xw = jnp.dot(x_ref[...], w_ref[...], preferred_element_type=jnp.float32)
    y_ref[...] = dinv * xw


def _bn_relu(aggp_ref, y_ref, dinv_ref, b_ref, g_ref, be_ref):
    dinv = dinv_ref[...]
    h = dinv * (aggp_ref[0, :N_NODES] + aggp_ref[1, :N_NODES] + y_ref[...])
    h = h + b_ref[...]
    mu = jnp.sum(h, axis=0, keepdims=True) * (1.0 / N_NODES)
    d = h - mu
    var = jnp.sum(d * d, axis=0, keepdims=True) * (1.0 / N_NODES)
    hn = g_ref[...] * d * lax.rsqrt(var + 1e-5) + be_ref[...]
    return jnp.maximum(hn, 0.0)


def _mid_body(aggp_ref, y_ref, dinv_ref, b_ref, g_ref, be_ref, w_ref, y2_ref):
    hn = _bn_relu(aggp_ref, y_ref, dinv_ref, b_ref, g_ref, be_ref)
    xw2 = jnp.dot(hn, w_ref[...], preferred_element_type=jnp.float32)
    y2_ref[...] = dinv_ref[...] * xw2


def _fin_body(aggp_ref, y_ref, dinv_ref, b_ref, g_ref, be_ref, wf_ref, bf_ref,
              out_ref):
    hn = _bn_relu(aggp_ref, y_ref, dinv_ref, b_ref, g_ref, be_ref)
    logits = jnp.dot(hn, wf_ref[...], preferred_element_type=jnp.float32)
    logits = logits + bf_ref[...]
    m = jnp.max(logits, axis=-1, keepdims=True)
    e = jnp.exp(logits - m)
    out_ref[...] = e / jnp.sum(e, axis=-1, keepdims=True)


_prep = pl.pallas_call(
    _prep_body,
    out_shape=(
        jax.ShapeDtypeStruct((N_NODES, 1), jnp.float32),
        jax.ShapeDtypeStruct((N_NODES, D_FEAT), jnp.float32),
    ),
)

_mid = pl.pallas_call(
    _mid_body,
    out_shape=jax.ShapeDtypeStruct((N_NODES, D_FEAT), jnp.float32),
)

_fin = pl.pallas_call(
    _fin_body,
    out_shape=jax.ShapeDtypeStruct((N_NODES, 10), jnp.float32),
)


def kernel(x, edge_index, W1, b1, W2, b2, W3, b3, g1, be1, g2, be2, g3, be3,
           Wf, bf):
    row = edge_index[0]
    col = edge_index[1]
    pad = E_PAD - N_EDGES
    rp = jnp.concatenate([row, jnp.zeros((pad,), jnp.int32)])
    cp = jnp.concatenate([col, jnp.full((pad,), DUMMY_COL, jnp.int32)])
    rp = rp.reshape(E_PAD // CHUNK, CHUNK)
    cp = cp.reshape(E_PAD // CHUNK, CHUNK)

    degp = _hist(cp)
    dinv, y1 = _prep(x, W1, degp)
    agg1 = _agg(y1, rp, cp)
    y2 = _mid(agg1, y1, dinv, b1.reshape(1, -1), g1.reshape(1, -1),
              be1.reshape(1, -1), W2)
    agg2 = _agg(y2, rp, cp)
    y3 = _mid(agg2, y2, dinv, b2.reshape(1, -1), g2.reshape(1, -1),
              be2.reshape(1, -1), W3)
    agg3 = _agg(y3, rp, cp)
    return _fin(agg3, y3, dinv, b3.reshape(1, -1), g3.reshape(1, -1),
                be3.reshape(1, -1), Wf, bf.reshape(1, -1))


# speedup vs baseline: 8.9345x; 1.0928x over previous
import functools

import jax
import jax.numpy as jnp
from jax import lax
from jax.experimental import pallas as pl
from jax.experimental.pallas import tpu as pltpu
from jax.experimental.pallas import tpu_sc as plsc

N_NODES = 10000
N_EDGES = 320000
D_FEAT = 128

NC = 2
NS = 16

CHUNK = 128
N_PAD = 10112
ROWS_PT = N_PAD // NS
E_PAD = 327680
NROWS = E_PAD // CHUNK
IDXROWS_PW = NROWS // (NC * NS)
CHUNKS_HIST = IDXROWS_PW
SUP = 8
NSUP0 = 17
NSUP1 = 3
CH0 = NSUP0 * SUP
CH1 = NSUP1 * SUP
DUMMY_COL = N_PAD - 8

_mesh = plsc.VectorSubcoreMesh(core_axis_name="c", subcore_axis_name="s")


def _memset2d(ref, nrows, ncols):
    zero16 = jnp.zeros((16,), jnp.float32)

    def row_body(r, _):
        for cc in range(ncols // 16):
            ref[r, pl.ds(cc * 16, 16)] = zero16
        return 0

    lax.fori_loop(0, nrows, row_body, 0)


@functools.partial(
    pl.kernel,
    mesh=_mesh,
    out_type=jax.ShapeDtypeStruct((NC, N_PAD, 16), jnp.float32),
    scratch_types=[
        pltpu.VMEM((IDXROWS_PW, CHUNK), jnp.int32),
        pltpu.VMEM((CHUNK, 16), jnp.float32),
        pltpu.VMEM_SHARED((N_PAD, 16), jnp.float32),
    ],
)
def _hist(col_hbm, out_hbm, cidx, ones_v, acc):
    c = lax.axis_index("c")
    s = lax.axis_index("s")
    wid = c * NS + s

    _memset2d(ones_v, CHUNK, 16)
    for r in range(ROWS_PT // CHUNK):
        pltpu.sync_copy(ones_v.at[pl.ds(0, CHUNK)],
                        acc.at[pl.ds(s * ROWS_PT + r * CHUNK, CHUNK)])
    rem = ROWS_PT % CHUNK
    pltpu.sync_copy(ones_v.at[pl.ds(0, rem)],
                    acc.at[pl.ds(s * ROWS_PT + (ROWS_PT // CHUNK) * CHUNK, rem)])

    one16 = jnp.ones((16,), jnp.float32)

    def fill_body(r, _):
        ones_v[r, pl.ds(0, 16)] = one16
        return 0

    lax.fori_loop(0, CHUNK, fill_body, 0)
    pltpu.sync_copy(col_hbm.at[pl.ds(wid * IDXROWS_PW, IDXROWS_PW)], cidx)
    plsc.subcore_barrier()

    def chunk_body(j, _):
        pltpu.sync_copy(ones_v, acc.at[cidx.at[j]], add=True)
        return 0

    lax.fori_loop(0, CHUNKS_HIST, chunk_body, 0)
    plsc.subcore_barrier()
    pltpu.sync_copy(acc.at[pl.ds(s * ROWS_PT, ROWS_PT)],
                    out_hbm.at[c, pl.ds(s * ROWS_PT, ROWS_PT)])


@functools.partial(
    pl.kernel,
    mesh=_mesh,
    out_type=jax.ShapeDtypeStruct((NC, N_PAD, D_FEAT), jnp.float32),
    scratch_types=[
        pltpu.VMEM((2, SUP, CHUNK), jnp.int32),
        pltpu.VMEM((2, SUP, CHUNK), jnp.int32),
        pltpu.VMEM((CHUNK, D_FEAT), jnp.float32),
        pltpu.VMEM((CHUNK, D_FEAT), jnp.float32),
        pltpu.VMEM_SHARED((N_PAD, D_FEAT), jnp.float32),
        pltpu.SemaphoreType.DMA,
        pltpu.SemaphoreType.DMA,
        pltpu.SemaphoreType.DMA,
        pltpu.SemaphoreType.DMA,
        pltpu.SemaphoreType.DMA,
    ],
)
def _agg(y_hbm, row_hbm, col_hbm, out_hbm, ridx, cidx, bufa, bufb, acc,
         sema, semb, semr, sca, scb):
    c = lax.axis_index("c")
    s = lax.axis_index("s")
    base = jnp.where(c == 0, s * CH0, NS * CH0 + s * CH1)
    nsup = jnp.where(c == 0, NSUP0, NSUP1)

    _memset2d(bufa, CHUNK, D_FEAT)
    for r in range(ROWS_PT // CHUNK):
        pltpu.sync_copy(bufa, acc.at[pl.ds(s * ROWS_PT + r * CHUNK, CHUNK)])
    rem = ROWS_PT % CHUNK
    pltpu.sync_copy(bufa.at[pl.ds(0, rem)],
                    acc.at[pl.ds(s * ROWS_PT + (ROWS_PT // CHUNK) * CHUNK, rem)])
    plsc.subcore_barrier()

    pltpu.sync_copy(row_hbm.at[pl.ds(base, SUP)], ridx.at[0])
    pltpu.sync_copy(col_hbm.at[pl.ds(base, SUP)], cidx.at[0])
    pltpu.async_copy(row_hbm.at[pl.ds(base + SUP, SUP)], ridx.at[1], semr)
    pltpu.async_copy(col_hbm.at[pl.ds(base + SUP, SUP)], cidx.at[1], semr)
    pltpu.async_copy(y_hbm.at[ridx.at[0, 0]], bufa, sema)

    def super_body(g, _):
        slot = g % 2
        nslot = (g + 1) % 2
        for j in range(SUP):
            cur, gcur = (bufa, sema) if j % 2 == 0 else (bufb, semb)
            nxt, gnxt = (bufb, semb) if j % 2 == 0 else (bufa, sema)
            scur = sca if j % 2 == 0 else scb
            snxt = scb if j % 2 == 0 else sca
            if j == 0:
                @pl.when(g > 0)
                def _():
                    pltpu.make_async_copy(
                        nxt, acc.at[cidx.at[nslot, SUP - 1]], snxt).wait()

                @pl.when(jnp.logical_and(g > 0, g < nsup - 1))
                def _():
                    b1 = base + (g + 1) * SUP
                    pltpu.async_copy(row_hbm.at[pl.ds(b1, SUP)],
                                     ridx.at[nslot], semr)
                    pltpu.async_copy(col_hbm.at[pl.ds(b1, SUP)],
                                     cidx.at[nslot], semr)
            else:
                pltpu.make_async_copy(
                    nxt, acc.at[cidx.at[slot, j - 1]], snxt).wait()
            if j + 1 < SUP:
                pltpu.async_copy(y_hbm.at[ridx.at[slot, j + 1]], nxt, gnxt)
            else:
                @pl.when(g < nsup - 1)
                def _():
                    nb = base + (g + 1) * SUP
                    pltpu.make_async_copy(
                        row_hbm.at[pl.ds(nb, SUP)], ridx.at[nslot],
                        semr).wait()
                    pltpu.make_async_copy(
                        col_hbm.at[pl.ds(nb, SUP)], cidx.at[nslot],
                        semr).wait()
                    pltpu.async_copy(y_hbm.at[ridx.at[nslot, 0]], nxt, gnxt)
            pltpu.make_async_copy(y_hbm.at[ridx.at[slot, j]], cur, gcur).wait()
            pltpu.async_copy(cur, acc.at[cidx.at[slot, j]], scur, add=True)
        return 0

    lax.fori_loop(0, nsup, super_body, 0)
    pltpu.make_async_copy(bufb, acc.at[cidx.at[1, SUP - 1]], scb).wait()

    plsc.subcore_barrier()
    pltpu.sync_copy(acc.at[pl.ds(s * ROWS_PT, ROWS_PT)],
                    out_hbm.at[c, pl.ds(s * ROWS_PT, ROWS_PT)])


def _prep_body(x_ref, w_ref, degp_ref, dinv_ref, y_ref):
    deg = degp_ref[0, :N_NODES, 0:1] + degp_ref[1, :N_NODES, 0:1] + 1.0
    dinv = lax.rsqrt(deg)
    dinv_ref[...] = dinv
    xw = jnp.dot(x_ref[...], w_ref[...], preferred_element_type=jnp.float32)
    y_ref[...] = dinv * xw


def _bn_relu(aggp_ref, y_ref, dinv_ref, b_ref, g_ref, be_ref):
    dinv = dinv_ref[...]
    h = dinv * (aggp_ref[0, :N_NODES] + aggp_ref[1, :N_NODES] + y_ref[...])
    h = h + b_ref[...]
    mu = jnp.sum(h, axis=0, keepdims=True) * (1.0 / N_NODES)
    d = h - mu
    var = jnp.sum(d * d, axis=0, keepdims=True) * (1.0 / N_NODES)
    hn = g_ref[...] * d * lax.rsqrt(var + 1e-5) + be_ref[...]
    return jnp.maximum(hn, 0.0)


def _mid_body(aggp_ref, y_ref, dinv_ref, b_ref, g_ref, be_ref, w_ref, y2_ref):
    hn = _bn_relu(aggp_ref, y_ref, dinv_ref, b_ref, g_ref, be_ref)
    xw2 = jnp.dot(hn, w_ref[...], preferred_element_type=jnp.float32)
    y2_ref[...] = dinv_ref[...] * xw2


def _fin_body(aggp_ref, y_ref, dinv_ref, b_ref, g_ref, be_ref, wf_ref, bf_ref,
              out_ref):
    hn = _bn_relu(aggp_ref, y_ref, dinv_ref, b_ref, g_ref, be_ref)
    logits = jnp.dot(hn, wf_ref[...], preferred_element_type=jnp.float32)
    logits = logits + bf_ref[...]
    m = jnp.max(logits, axis=-1, keepdims=True)
    e = jnp.exp(logits - m)
    out_ref[...] = e / jnp.sum(e, axis=-1, keepdims=True)


_prep = pl.pallas_call(
    _prep_body,
    out_shape=(
        jax.ShapeDtypeStruct((N_NODES, 1), jnp.float32),
        jax.ShapeDtypeStruct((N_NODES, D_FEAT), jnp.float32),
    ),
)

_mid = pl.pallas_call(
    _mid_body,
    out_shape=jax.ShapeDtypeStruct((N_NODES, D_FEAT), jnp.float32),
)

_fin = pl.pallas_call(
    _fin_body,
    out_shape=jax.ShapeDtypeStruct((N_NODES, 10), jnp.float32),
)


def kernel(x, edge_index, W1, b1, W2, b2, W3, b3, g1, be1, g2, be2, g3, be3,
           Wf, bf):
    row = edge_index[0]
    col = edge_index[1]
    pad = E_PAD - N_EDGES
    rp = jnp.concatenate([row, jnp.zeros((pad,), jnp.int32)])
    cp = jnp.concatenate([col, jnp.full((pad,), DUMMY_COL, jnp.int32)])
    rp = rp.reshape(NROWS, CHUNK)
    cp = cp.reshape(NROWS, CHUNK)

    degp = _hist(cp)
    dinv, y1 = _prep(x, W1, degp)
    agg1 = _agg(y1, rp, cp)
    y2 = _mid(agg1, y1, dinv, b1.reshape(1, -1), g1.reshape(1, -1),
              be1.reshape(1, -1), W2)
    agg2 = _agg(y2, rp, cp)
    y3 = _mid(agg2, y2, dinv, b2.reshape(1, -1), g2.reshape(1, -1),
              be2.reshape(1, -1), W3)
    agg3 = _agg(y3, rp, cp)
    return _fin(agg3, y3, dinv, b3.reshape(1, -1), g3.reshape(1, -1),
                be3.reshape(1, -1), Wf, bf.reshape(1, -1))
